# baseline (speedup 1.0000x reference)
import functools

import jax
import jax.numpy as jnp
from jax import lax
from jax.experimental import pallas as pl
from jax.experimental.pallas import tpu as pltpu
from jax.experimental.pallas import tpu_sc as plsc

D_MODEL = 64
SEQ = 200
NX = 4096
NROWS = NX * SEQ
IDXW = 128
N_IROWS = NROWS // IDXW
NW = 32
IRPW = N_IROWS // NW
RPW = NROWS // NW
G_PER_STEP = 2
CH = G_PER_STEP * IDXW
N_STEP = IRPW // G_PER_STEP
N_TABLE = 8192
TROWS = N_TABLE // 16


def _make_gather():
  mesh = plsc.VectorSubcoreMesh(
      core_axis_name="c", subcore_axis_name="s", num_cores=2, num_subcores=16
  )

  @functools.partial(
      pl.kernel,
      mesh=mesh,
      compiler_params=pltpu.CompilerParams(use_tc_tiling_on_sc=False),
      out_type=jax.ShapeDtypeStruct((NROWS, D_MODEL), jnp.float32),
      scratch_types=[
          pltpu.VMEM_SHARED((N_TABLE, D_MODEL), jnp.float32),
          pltpu.VMEM((IRPW, IDXW), jnp.int32),
          pltpu.VMEM((CH, D_MODEL), jnp.float32),
          pltpu.VMEM((CH, D_MODEL), jnp.float32),
          pltpu.VMEM((CH, D_MODEL), jnp.float32),
          pltpu.SemaphoreType.DMA,
          pltpu.SemaphoreType.DMA,
          pltpu.SemaphoreType.DMA,
          pltpu.SemaphoreType.DMA,
          pltpu.SemaphoreType.DMA,
          pltpu.SemaphoreType.DMA,
      ],
  )
  def gather_kernel(
      x_hbm, pe_hbm, out_hbm, pe_sh, idx_v, buf_a, buf_b, buf_c,
      gsem_a, gsem_b, gsem_c, osem_a, osem_b, osem_c
  ):
    sid = lax.axis_index("s")
    wid = sid * 2 + lax.axis_index("c")
    orow0 = wid * RPW

    pltpu.sync_copy(
        pe_hbm.at[pl.ds(sid * TROWS, TROWS)],
        pe_sh.at[pl.ds(sid * TROWS, TROWS)],
    )
    pltpu.sync_copy(x_hbm.at[pl.ds(wid * IRPW, IRPW)], idx_v)
    plsc.subcore_barrier()

    def issue_gathers(s, buf, gsem):
      for k in range(G_PER_STEP):
        pltpu.async_copy(
            pe_sh.at[idx_v.at[s * G_PER_STEP + k]],
            buf.at[pl.ds(k * IDXW, IDXW)],
            gsem,
        )

    def wait_gathers(s, buf, gsem):
      for k in range(G_PER_STEP):
        pltpu.make_async_copy(
            pe_sh.at[idx_v.at[s * G_PER_STEP + k]],
            buf.at[pl.ds(k * IDXW, IDXW)],
            gsem,
        ).wait()

    def issue_out(s, buf, osem):
      pltpu.async_copy(buf, out_hbm.at[pl.ds(orow0 + s * CH, CH)], osem)

    def wait_out(s, buf, osem):
      pltpu.make_async_copy(
          buf, out_hbm.at[pl.ds(orow0 + s * CH, CH)], osem
      ).wait()

    bufs = (buf_a, buf_b, buf_c)
    gsems = (gsem_a, gsem_b, gsem_c)
    osems = (osem_a, osem_b, osem_c)

    issue_gathers(0, buf_a, gsem_a)
    issue_gathers(1, buf_b, gsem_b)

    def step(s, carry):
      def body(p):
        pn = (p + 2) % 3
        wait_gathers(s, bufs[p], gsems[p])
        issue_out(s, bufs[p], osems[p])

        @pl.when(s < N_STEP - 2)
        def _():
          @pl.when(s > 0)
          def _():
            wait_out(s - 1, bufs[pn], osems[pn])

          issue_gathers(s + 2, bufs[pn], gsems[pn])

      phase = s % 3
      for p in range(3):
        @pl.when(phase == p)
        def _(p=p):
          body(p)

      return carry

    lax.fori_loop(0, N_STEP, step, 0)

    for t in (N_STEP - 3, N_STEP - 2, N_STEP - 1):
      wait_out(t, bufs[t % 3], osems[t % 3])

  return gather_kernel


def kernel(x, pe):
  xf = x.astype(jnp.int32).reshape(N_IROWS, IDXW)
  out = _make_gather()(xf, pe)
  return out.reshape(NX, SEQ, D_MODEL)
